# baseline (speedup 1.0000x reference)
import functools

import jax
import jax.numpy as jnp
from jax import lax
from jax.experimental import pallas as pl
from jax.experimental.pallas import tpu as pltpu
from jax.experimental.pallas import tpu_sc as plsc

D = 64
V = 100000
HIST = 168
PRED = 24
NF_HIST = 7
NF_FUT = 3
IDXW = 128
CHUNK = 3


def _gather_all(tab, idx_h, idx_f, idx_s, nw):
    rows_h = idx_h.shape[0]
    rows_f = idx_f.shape[0]
    rows_s = idx_s.shape[0]
    crows = CHUNK * IDXW
    h_rows_w = rows_h // nw
    f_rows_w = rows_f // nw
    h_chunks = h_rows_w // crows
    f_chunks = f_rows_w // crows
    s_per_w = rows_s // nw

    mesh = plsc.VectorSubcoreMesh(core_axis_name="c", subcore_axis_name="s")
    nc = mesh.num_cores

    @functools.partial(
        pl.kernel,
        out_type=[
            jax.ShapeDtypeStruct((rows_h, D), jnp.float32),
            jax.ShapeDtypeStruct((rows_f, D), jnp.float32),
            jax.ShapeDtypeStruct((rows_s, D), jnp.float32),
        ],
        mesh=mesh,
        compiler_params=pltpu.CompilerParams(use_tc_tiling_on_sc=False),
        scratch_types=[
            pltpu.VMEM((crows,), jnp.int32),
            pltpu.VMEM((crows, D), jnp.float32),
            pltpu.VMEM((s_per_w,), jnp.int32),
            pltpu.VMEM((s_per_w, D), jnp.float32),
            pltpu.SemaphoreType.DMA,
        ],
    )
    def k(tab_hbm, ih_hbm, if_hbm, is_hbm, oh_hbm, of_hbm, os_hbm,
          idx_v, rows_v, sidx_v, srows_v, sem):
        wid = lax.axis_index("s") * nc + lax.axis_index("c")

        def phase(idx1_hbm, out_hbm, n_chunks, row_base):
            def chunk_body(c, carry):
                b = row_base + c * crows
                pltpu.sync_copy(idx1_hbm.at[pl.ds(b, crows)], idx_v)
                cps = [
                    pltpu.async_copy(
                        tab_hbm.at[idx_v.at[pl.ds(j * IDXW, IDXW)]],
                        rows_v.at[pl.ds(j * IDXW, IDXW)],
                        sem,
                    )
                    for j in range(CHUNK)
                ]
                for cp in cps:
                    cp.wait()
                pltpu.sync_copy(rows_v, out_hbm.at[pl.ds(b, crows)])
                return carry
            lax.fori_loop(0, n_chunks, chunk_body, 0)

        phase(ih_hbm, oh_hbm, h_chunks, wid * h_rows_w)
        phase(if_hbm, of_hbm, f_chunks, wid * f_rows_w)

        pltpu.sync_copy(is_hbm.at[pl.ds(wid * s_per_w, s_per_w)], sidx_v)
        pltpu.async_copy(tab_hbm.at[sidx_v], srows_v, sem).wait()
        pltpu.sync_copy(srows_v, os_hbm.at[pl.ds(wid * s_per_w, s_per_w)])

    return k(tab, idx_h, idx_f, idx_s)


def kernel(inputs, tables):
    B, W, NI = inputs.shape
    total_window = HIST + PRED
    if W > total_window:
        inputs = inputs[:, -total_window:, :]
    inputs = inputs.astype(jnp.int32)
    tab = tables.reshape(NI * V, D)

    offs = (jnp.arange(1, NF_HIST + 1, dtype=jnp.int32) * V)
    idx_h = (inputs[:, :HIST, 1:1 + NF_HIST] + offs).reshape(-1)
    idx_f = (inputs[:, HIST:, 1:1 + NF_FUT] + offs[:NF_FUT]).reshape(-1)

    info = plsc.get_sparse_core_info()
    nw = info.num_cores * info.num_subcores
    idx_s = inputs[:, 0, 0]

    oh, of, os_ = _gather_all(tab, idx_h, idx_f, idx_s, nw)
    static = os_.reshape(B, 1, D)
    historical = oh.reshape(B, HIST, NF_HIST, D)
    future = of.reshape(B, PRED, NF_FUT, D)
    return (static, historical, future)


# speedup vs baseline: 1.0560x; 1.0560x over previous
import functools

import jax
import jax.numpy as jnp
from jax import lax
from jax.experimental import pallas as pl
from jax.experimental.pallas import tpu as pltpu
from jax.experimental.pallas import tpu_sc as plsc

D = 64
V = 100000
HIST = 168
PRED = 24
NF_HIST = 7
NF_FUT = 3
LANES = 128
QROWS = 256


def _gather_all(tab, idx4, nw):
    rows_h = HIST * NF_HIST * 1024
    rows_f = PRED * NF_FUT * 1024
    rows_s = 1024
    hq = rows_h // QROWS // nw
    fq = rows_f // QROWS // nw
    s_per_w = rows_s // nw

    mesh = plsc.VectorSubcoreMesh(core_axis_name="c", subcore_axis_name="s")
    nc = mesh.num_cores

    @functools.partial(
        pl.kernel,
        out_type=[
            jax.ShapeDtypeStruct((rows_h, D), jnp.float32),
            jax.ShapeDtypeStruct((rows_f, D), jnp.float32),
            jax.ShapeDtypeStruct((rows_s, D), jnp.float32),
        ],
        mesh=mesh,
        compiler_params=pltpu.CompilerParams(use_tc_tiling_on_sc=False),
        scratch_types=[
            pltpu.VMEM((2, LANES), jnp.int32),
            pltpu.VMEM((QROWS, D), jnp.float32),
            pltpu.VMEM((s_per_w,), jnp.int32),
            pltpu.VMEM((s_per_w, D), jnp.float32),
            pltpu.SemaphoreType.DMA,
        ],
    )
    def k(tab_hbm, idx_hbm, oh_hbm, of_hbm, os_hbm,
          idx_v, rows_v, sidx_v, srows_v, sem):
        wid = lax.axis_index("s") * nc + lax.axis_index("c")

        def phase(out_hbm, n_chunks, nf, t0):
            blocks_per_t = nf * 4

            def chunk_body(c, carry):
                q = wid * n_chunks + c
                t = t0 + q // blocks_per_t
                rem = q % blocks_per_t
                f = 1 + rem // 4
                quarter = rem % 4
                pltpu.sync_copy(
                    idx_hbm.at[t, pl.ds(quarter * 2, 2), f, :], idx_v)
                cps = [
                    pltpu.async_copy(
                        tab_hbm.at[idx_v.at[j]],
                        rows_v.at[pl.ds(j * LANES, LANES)],
                        sem,
                    )
                    for j in range(2)
                ]
                for cp in cps:
                    cp.wait()
                pltpu.sync_copy(rows_v, out_hbm.at[pl.ds(q * QROWS, QROWS)])
                return carry

            lax.fori_loop(0, n_chunks, chunk_body, 0)

        phase(oh_hbm, hq, NF_HIST, 0)
        phase(of_hbm, fq, NF_FUT, HIST)

        pltpu.sync_copy(
            idx_hbm.at[0, wid // 4, 0, pl.ds((wid % 4) * s_per_w, s_per_w)],
            sidx_v)
        pltpu.async_copy(tab_hbm.at[sidx_v], srows_v, sem).wait()
        pltpu.sync_copy(srows_v, os_hbm.at[pl.ds(wid * s_per_w, s_per_w)])

    return k(tab, idx4)


def kernel(inputs, tables):
    B, W, NI = inputs.shape
    total_window = HIST + PRED
    if W > total_window:
        inputs = inputs[:, -total_window:, :]
        W = total_window
    inputs = inputs.astype(jnp.int32)
    tab = tables.reshape(NI * V, D)

    x = inputs.reshape(B // LANES, LANES, W, NI).transpose(2, 0, 3, 1)
    idx4 = x + (jnp.arange(NI, dtype=jnp.int32) * V)[None, None, :, None]

    info = plsc.get_sparse_core_info()
    nw = info.num_cores * info.num_subcores

    oh, of, os_ = _gather_all(tab, idx4, nw)
    static = os_.reshape(B, 1, D)
    historical = oh.reshape(HIST, NF_HIST, B, D).transpose(2, 0, 1, 3)
    future = of.reshape(PRED, NF_FUT, B, D).transpose(2, 0, 1, 3)
    return (static, historical, future)
